# baseline (speedup 1.0000x reference)
import jax
import jax.numpy as jnp
from jax import lax
from jax.experimental import pallas as pl
from jax.experimental.pallas import tpu as pltpu
from jax.experimental.pallas import tpu_sc as plsc

N = 10000
E = 320000
NP = 10240
NW = 32
CK = 128
NCH = 80
NCH2 = 160
EP = NW * NCH * CK
RPT = NP // 16
DHQ = 32
D2 = 16

_mesh = plsc.VectorSubcoreMesh(core_axis_name="c", subcore_axis_name="s")
_sc_params = pltpu.CompilerParams(
    needs_layout_passes=False, use_tc_tiling_on_sc=False
)


def _tc_lin1_body(x_ref, w_ref, a_ref, hlo_ref, hhi_ref, h2lo_ref, h2hi_ref,
                  aa_ref):
    h = jnp.dot(x_ref[...], w_ref[...], preferred_element_type=jnp.float32)
    hlo_ref[...] = h[:, :DHQ]
    hhi_ref[...] = h[:, DHQ:2 * DHQ]
    h2lo_ref[...] = h[:, 2 * DHQ:3 * DHQ]
    h2hi_ref[...] = h[:, 3 * DHQ:]
    aa_ref[...] = jnp.dot(h, a_ref[...], preferred_element_type=jnp.float32)


def _tc_lin1(x, w, a):
    return pl.pallas_call(
        _tc_lin1_body,
        grid=(10,),
        in_specs=[
            pl.BlockSpec((1000, 128), lambda i: (i, 0)),
            pl.BlockSpec((128, 128), lambda i: (0, 0)),
            pl.BlockSpec((128, 2), lambda i: (0, 0)),
        ],
        out_specs=[
            pl.BlockSpec((1000, DHQ), lambda i: (i, 0)),
            pl.BlockSpec((1000, DHQ), lambda i: (i, 0)),
            pl.BlockSpec((1000, DHQ), lambda i: (i, 0)),
            pl.BlockSpec((1000, DHQ), lambda i: (i, 0)),
            pl.BlockSpec((1000, 2), lambda i: (i, 0)),
        ],
        out_shape=[
            jax.ShapeDtypeStruct((N, DHQ), jnp.float32),
            jax.ShapeDtypeStruct((N, DHQ), jnp.float32),
            jax.ShapeDtypeStruct((N, DHQ), jnp.float32),
            jax.ShapeDtypeStruct((N, DHQ), jnp.float32),
            jax.ShapeDtypeStruct((N, 2), jnp.float32),
        ],
    )(x, w, a)


def _tc_lin2_body(p0_ref, p1_ref, p2_ref, p3_ref, b_ref, w_ref, a_ref,
                  h2_ref, aa_ref):
    g = jnp.concatenate(
        [p0_ref[...], p1_ref[...], p2_ref[...], p3_ref[...]], axis=1)
    g = jnp.maximum(g + b_ref[...], 0.0)
    h2 = jnp.dot(g, w_ref[...], preferred_element_type=jnp.float32)
    h2_ref[...] = h2
    aa_ref[...] = jnp.dot(h2, a_ref[...], preferred_element_type=jnp.float32)


def _tc_lin2(p0, p1, p2, p3, b1, w2, a2):
    return pl.pallas_call(
        _tc_lin2_body,
        grid=(10,),
        in_specs=[
            pl.BlockSpec((1024, DHQ), lambda i: (i, 0)),
            pl.BlockSpec((1024, DHQ), lambda i: (i, 0)),
            pl.BlockSpec((1024, DHQ), lambda i: (i, 0)),
            pl.BlockSpec((1024, DHQ), lambda i: (i, 0)),
            pl.BlockSpec((1, 128), lambda i: (0, 0)),
            pl.BlockSpec((128, D2), lambda i: (0, 0)),
            pl.BlockSpec((D2, 2), lambda i: (0, 0)),
        ],
        out_specs=[
            pl.BlockSpec((1024, D2), lambda i: (i, 0)),
            pl.BlockSpec((1024, 2), lambda i: (i, 0)),
        ],
        out_shape=[
            jax.ShapeDtypeStruct((NP, D2), jnp.float32),
            jax.ShapeDtypeStruct((NP, 2), jnp.float32),
        ],
    )(p0, p1, p2, p3, b1, w2, a2)


def _tc_fin_body(q0_ref, q1_ref, b_ref, o_ref):
    o_ref[...] = q0_ref[...] + q1_ref[...] + b_ref[...]


def _tc_fin(q0, q1, b2):
    return pl.pallas_call(
        _tc_fin_body,
        grid=(10,),
        in_specs=[
            pl.BlockSpec((1024, D2), lambda i: (i, 0)),
            pl.BlockSpec((1024, D2), lambda i: (i, 0)),
            pl.BlockSpec((1, D2), lambda i: (0, 0)),
        ],
        out_specs=pl.BlockSpec((1024, D2), lambda i: (i, 0)),
        out_shape=jax.ShapeDtypeStruct((NP, D2), jnp.float32),
    )(q0, q1, b2)


def _zero_slice(z_v, sh, sid):
    @pl.loop(0, RPT // 16)
    def _zero(i):
        z_v[pl.ds(i * 16, 16)] = jnp.zeros((16,), jnp.float32)

    pltpu.sync_copy(z_v, sh.at[pl.ds(sid * RPT, RPT)])


def _compute_ex(as_v, ad_v, src_v, dst_v, ex_v):
    @pl.loop(0, NCH2)
    def _chunk(j):
        for k in range(CK // 16):
            ix = pl.ds(k * 16, 16)
            sv = src_v[j, ix]
            dv = dst_v[j, ix]
            e = plsc.load_gather(as_v, [sv]) + plsc.load_gather(ad_v, [dv])
            e = jnp.where(e >= 0.0, e, e * 0.2)
            ex_v[j, ix] = jnp.exp(e)


def _scatter_ex(ex_v, dst_v, s_sh, xsem):
    @pl.loop(0, NCH2, step=16)
    def _scat(j0):
        for b in range(16):
            pltpu.async_copy(ex_v.at[j0 + b], s_sh.at[dst_v.at[j0 + b]],
                             xsem, add=True)
        for b in range(16):
            pltpu.make_async_copy(ex_v.at[j0 + b],
                                  s_sh.at[dst_v.at[j0 + b]], xsem).wait()


def _zero_rows(rows, D):
    @pl.loop(0, CK)
    def _zrow(e):
        for k in range(D // 16):
            rows[0, e, pl.ds(k * 16, 16)] = jnp.zeros((16,), jnp.float32)


def _pipeline_chunks(D, j_lo, j_hi, h_hbm, src_v, dst_v, al_v, rows, out_sh,
                     gsem, ssem):
    def g_issue(j, b):
        pltpu.async_copy(h_hbm.at[src_v.at[j]], rows.at[b], gsem)

    def g_wait(j, b):
        pltpu.make_async_copy(h_hbm.at[src_v.at[j]], rows.at[b], gsem).wait()

    def s_issue(j, b):
        pltpu.async_copy(rows.at[b], out_sh.at[dst_v.at[j]], ssem, add=True)

    def s_wait(j, b):
        pltpu.make_async_copy(rows.at[b], out_sh.at[dst_v.at[j]], ssem).wait()

    g_issue(j_lo + 0, 0)
    g_issue(j_lo + 1, 1)
    g_issue(j_lo + 2, 2)

    @pl.loop(j_lo, j_hi, step=4)
    def _steps(j0):
        for b in range(4):
            j = j0 + b
            g_wait(j, b)

            @pl.loop(0, CK // 16)
            def _scale(g):
                av = al_v[j, pl.ds(g * 16, 16)]
                for l in range(16):
                    a = av[l]
                    e = g * 16 + l
                    for k in range(D // 16):
                        ix = pl.ds(k * 16, 16)
                        rows[b, e, ix] = rows[b, e, ix] * a

            @pl.when(j > j_lo)
            def _drain_prev():
                s_wait(j, b)

            s_issue(j, b)

            @pl.when(j + 3 < j_hi)
            def _prefetch():
                g_issue(j + 3, (b + 3) % 4)

    s_wait(j_lo, 0)


def _sc_l1_body(src_hbm, dst_hbm, as_hbm, ad_hbm, h_hbm,
                q00_hbm, q01_hbm, q10_hbm, q11_hbm,
                as_v, ad_v, s_v, src_v, dst_v, ex_v, z_v, rows,
                s_sh, out_sh, gsem, ssem, xsem):
    cid = lax.axis_index("c")
    sid = lax.axis_index("s")

    _zero_slice(z_v, s_sh, sid)

    pltpu.sync_copy(as_hbm, as_v)
    pltpu.sync_copy(ad_hbm, ad_v)
    pltpu.sync_copy(src_hbm.at[sid], src_v)
    pltpu.sync_copy(dst_hbm.at[sid], dst_v)

    _compute_ex(as_v, ad_v, src_v, dst_v, ex_v)
    plsc.subcore_barrier()
    _scatter_ex(ex_v, dst_v, s_sh, xsem)
    plsc.subcore_barrier()
    pltpu.sync_copy(s_sh, s_v)

    off = cid * (2 * NP)

    @pl.loop(0, NCH2)
    def _alpha(j):
        for k in range(CK // 16):
            ix = pl.ds(k * 16, 16)
            dv = dst_v[j, ix]
            sg = plsc.load_gather(s_v, [dv])
            ex_v[j, ix] = ex_v[j, ix] / (sg + 1e-16)
            src_v[j, ix] = src_v[j, ix] + off

    quarters = ((q00_hbm, q10_hbm), (q01_hbm, q11_hbm))
    for p in range(2):
        _zero_rows(rows, DHQ)
        for r in range(RPT // CK):
            pltpu.sync_copy(rows.at[0],
                            out_sh.at[pl.ds(sid * RPT + r * CK, CK)])
        plsc.subcore_barrier()

        _pipeline_chunks(DHQ, 0, NCH2, h_hbm, src_v, dst_v, ex_v, rows,
                         out_sh, gsem, ssem)
        plsc.subcore_barrier()

        qa, qb = quarters[p]
        for r in range(RPT // CK):
            sl = pl.ds(sid * RPT + r * CK, CK)

            @pl.when(cid == 0)
            def _c0():
                pltpu.sync_copy(out_sh.at[sl], qa.at[sl])

            @pl.when(cid == 1)
            def _c1():
                pltpu.sync_copy(out_sh.at[sl], qb.at[sl])

        if p == 0:
            @pl.loop(0, NCH2)
            def _bump(j):
                for k in range(CK // 16):
                    ix = pl.ds(k * 16, 16)
                    src_v[j, ix] = src_v[j, ix] + NP


_sc_l1 = pl.kernel(
    _sc_l1_body,
    out_type=[
        jax.ShapeDtypeStruct((NP, DHQ), jnp.float32),
        jax.ShapeDtypeStruct((NP, DHQ), jnp.float32),
        jax.ShapeDtypeStruct((NP, DHQ), jnp.float32),
        jax.ShapeDtypeStruct((NP, DHQ), jnp.float32),
    ],
    mesh=_mesh,
    compiler_params=_sc_params,
    scratch_types=[
        pltpu.VMEM((NP,), jnp.float32),
        pltpu.VMEM((NP,), jnp.float32),
        pltpu.VMEM((NP,), jnp.float32),
        pltpu.VMEM((NCH2, CK), jnp.int32),
        pltpu.VMEM((NCH2, CK), jnp.int32),
        pltpu.VMEM((NCH2, CK), jnp.float32),
        pltpu.VMEM((RPT,), jnp.float32),
        pltpu.VMEM((4, CK, DHQ), jnp.float32),
        pltpu.VMEM_SHARED((NP,), jnp.float32),
        pltpu.VMEM_SHARED((NP, DHQ), jnp.float32),
        pltpu.SemaphoreType.DMA,
        pltpu.SemaphoreType.DMA,
        pltpu.SemaphoreType.DMA,
    ],
)


def _sc_l2_body(src_hbm, dst_hbm, as_hbm, ad_hbm, h_hbm,
                q0_hbm, q1_hbm,
                as_v, ad_v, s_v, src_v, dst_v, ex_v, z_v, rows,
                s_sh, out_sh, gsem, ssem, xsem):
    cid = lax.axis_index("c")
    sid = lax.axis_index("s")

    _zero_slice(z_v, s_sh, sid)

    pltpu.sync_copy(as_hbm, as_v)
    pltpu.sync_copy(ad_hbm, ad_v)
    pltpu.sync_copy(src_hbm.at[sid], src_v)
    pltpu.sync_copy(dst_hbm.at[sid], dst_v)

    _compute_ex(as_v, ad_v, src_v, dst_v, ex_v)
    plsc.subcore_barrier()
    _scatter_ex(ex_v, dst_v, s_sh, xsem)
    plsc.subcore_barrier()
    pltpu.sync_copy(s_sh, s_v)

    j_lo = cid * NCH
    j_hi = j_lo + NCH

    @pl.loop(j_lo, j_hi)
    def _alpha(j):
        for k in range(CK // 16):
            ix = pl.ds(k * 16, 16)
            dv = dst_v[j, ix]
            sg = plsc.load_gather(s_v, [dv])
            ex_v[j, ix] = ex_v[j, ix] / (sg + 1e-16)

    _zero_rows(rows, D2)
    for r in range(RPT // CK):
        pltpu.sync_copy(rows.at[0], out_sh.at[pl.ds(sid * RPT + r * CK, CK)])
    plsc.subcore_barrier()

    _pipeline_chunks(D2, j_lo, j_hi, h_hbm, src_v, dst_v, ex_v, rows,
                     out_sh, gsem, ssem)
    plsc.subcore_barrier()

    for r in range(RPT // CK):
        sl = pl.ds(sid * RPT + r * CK, CK)

        @pl.when(cid == 0)
        def _c0():
            pltpu.sync_copy(out_sh.at[sl], q0_hbm.at[sl])

        @pl.when(cid == 1)
        def _c1():
            pltpu.sync_copy(out_sh.at[sl], q1_hbm.at[sl])


_sc_l2 = pl.kernel(
    _sc_l2_body,
    out_type=[
        jax.ShapeDtypeStruct((NP, D2), jnp.float32),
        jax.ShapeDtypeStruct((NP, D2), jnp.float32),
    ],
    mesh=_mesh,
    compiler_params=_sc_params,
    scratch_types=[
        pltpu.VMEM((NP,), jnp.float32),
        pltpu.VMEM((NP,), jnp.float32),
        pltpu.VMEM((NP,), jnp.float32),
        pltpu.VMEM((NCH2, CK), jnp.int32),
        pltpu.VMEM((NCH2, CK), jnp.int32),
        pltpu.VMEM((NCH2, CK), jnp.float32),
        pltpu.VMEM((RPT,), jnp.float32),
        pltpu.VMEM((4, CK, D2), jnp.float32),
        pltpu.VMEM_SHARED((NP,), jnp.float32),
        pltpu.VMEM_SHARED((NP, D2), jnp.float32),
        pltpu.SemaphoreType.DMA,
        pltpu.SemaphoreType.DMA,
        pltpu.SemaphoreType.DMA,
    ],
)


def kernel(x, edge_index, W1, a_src1, a_dst1, b1, W2, a_src2, a_dst2, b2):
    src = edge_index[0]
    dst = edge_index[1]
    pad_ids = (jnp.arange(EP - E, dtype=jnp.int32) % (NP - N)) + N
    src16 = jnp.concatenate([src, pad_ids]).reshape(16, NCH2, CK)
    dst16 = jnp.concatenate([dst, pad_ids]).reshape(16, NCH2, CK)

    a1 = jnp.stack([a_src1, a_dst1], axis=1)
    h0, h1, h2q, h3, aa1 = _tc_lin1(x, W1, a1)
    htab1 = jnp.concatenate([
        jnp.pad(h0, ((0, NP - N), (0, 0))),
        jnp.pad(h1, ((0, NP - N), (0, 0))),
        jnp.pad(h2q, ((0, NP - N), (0, 0))),
        jnp.pad(h3, ((0, NP - N), (0, 0))),
    ])
    as1 = jnp.pad(aa1[:, 0], (0, NP - N))
    ad1 = jnp.pad(aa1[:, 1], (0, NP - N))

    q00, q01, q10, q11 = _sc_l1(src16, dst16, as1, ad1, htab1)

    b1r = b1.reshape(1, 128)
    w2p = jnp.pad(W2, ((0, 0), (0, D2 - 2)))
    a2 = jnp.pad(jnp.stack([a_src2, a_dst2], axis=1), ((0, D2 - 2), (0, 0)))
    h2, aa2 = _tc_lin2(q00, q01, q10, q11, b1r, w2p, a2)
    as2 = aa2[:, 0]
    ad2 = aa2[:, 1]

    q2a, q2b = _sc_l2(src16, dst16, as2, ad2, h2)

    b2r = jnp.pad(b2, (0, D2 - 2)).reshape(1, D2)
    out = _tc_fin(q2a, q2b, b2r)
    return out[:N, :2]


# speedup vs baseline: 1.0046x; 1.0046x over previous
import jax
import jax.numpy as jnp
from jax import lax
from jax.experimental import pallas as pl
from jax.experimental.pallas import tpu as pltpu
from jax.experimental.pallas import tpu_sc as plsc

N = 10000
E = 320000
NP = 10240
NW = 32
CK = 128
NCH = 80
NCH2 = 160
EP = NW * NCH * CK
RPT = NP // 16
DHQ = 32
D2 = 16

_mesh = plsc.VectorSubcoreMesh(core_axis_name="c", subcore_axis_name="s")
_sc_params = pltpu.CompilerParams(
    needs_layout_passes=False, use_tc_tiling_on_sc=False
)


def _tc_lin1_body(x_ref, w_ref, a_ref, h_ref, aa_ref):
    h = jnp.dot(x_ref[...], w_ref[...], preferred_element_type=jnp.float32)
    h_ref[...] = h
    aa_ref[...] = jnp.dot(h, a_ref[...], preferred_element_type=jnp.float32)


def _tc_lin1(x, w, a):
    return pl.pallas_call(
        _tc_lin1_body,
        grid=(10,),
        in_specs=[
            pl.BlockSpec((1000, 128), lambda i: (i, 0)),
            pl.BlockSpec((128, 128), lambda i: (0, 0)),
            pl.BlockSpec((128, 2), lambda i: (0, 0)),
        ],
        out_specs=[
            pl.BlockSpec((1000, 128), lambda i: (i, 0)),
            pl.BlockSpec((1000, 2), lambda i: (i, 0)),
        ],
        out_shape=[
            jax.ShapeDtypeStruct((N, 128), jnp.float32),
            jax.ShapeDtypeStruct((N, 2), jnp.float32),
        ],
    )(x, w, a)


def _tc_lin2_body(p0_ref, p1_ref, p2_ref, p3_ref, b_ref, w_ref, a_ref,
                  h2_ref, aa_ref):
    g = jnp.concatenate(
        [p0_ref[...], p1_ref[...], p2_ref[...], p3_ref[...]], axis=1)
    g = jnp.maximum(g + b_ref[...], 0.0)
    h2 = jnp.dot(g, w_ref[...], preferred_element_type=jnp.float32)
    h2_ref[...] = h2
    aa_ref[...] = jnp.dot(h2, a_ref[...], preferred_element_type=jnp.float32)


def _tc_lin2(p0, p1, p2, p3, b1, w2, a2):
    return pl.pallas_call(
        _tc_lin2_body,
        grid=(10,),
        in_specs=[
            pl.BlockSpec((1024, DHQ), lambda i: (i, 0)),
            pl.BlockSpec((1024, DHQ), lambda i: (i, 0)),
            pl.BlockSpec((1024, DHQ), lambda i: (i, 0)),
            pl.BlockSpec((1024, DHQ), lambda i: (i, 0)),
            pl.BlockSpec((1, 128), lambda i: (0, 0)),
            pl.BlockSpec((128, D2), lambda i: (0, 0)),
            pl.BlockSpec((D2, 2), lambda i: (0, 0)),
        ],
        out_specs=[
            pl.BlockSpec((1024, D2), lambda i: (i, 0)),
            pl.BlockSpec((1024, 2), lambda i: (i, 0)),
        ],
        out_shape=[
            jax.ShapeDtypeStruct((NP, D2), jnp.float32),
            jax.ShapeDtypeStruct((NP, 2), jnp.float32),
        ],
    )(p0, p1, p2, p3, b1, w2, a2)


def _tc_fin_body(q0_ref, q1_ref, b_ref, o_ref):
    o_ref[...] = q0_ref[...] + q1_ref[...] + b_ref[...]


def _tc_fin(q0, q1, b2):
    return pl.pallas_call(
        _tc_fin_body,
        grid=(10,),
        in_specs=[
            pl.BlockSpec((1024, D2), lambda i: (i, 0)),
            pl.BlockSpec((1024, D2), lambda i: (i, 0)),
            pl.BlockSpec((1, D2), lambda i: (0, 0)),
        ],
        out_specs=pl.BlockSpec((1024, D2), lambda i: (i, 0)),
        out_shape=jax.ShapeDtypeStruct((NP, D2), jnp.float32),
    )(q0, q1, b2)


def _zero_slice(z_v, sh, sid):
    @pl.loop(0, RPT // 16)
    def _zero(i):
        z_v[pl.ds(i * 16, 16)] = jnp.zeros((16,), jnp.float32)

    pltpu.sync_copy(z_v, sh.at[pl.ds(sid * RPT, RPT)])


def _compute_ex(as_v, ad_v, src_v, dst_v, ex_v):
    @pl.loop(0, NCH2)
    def _chunk(j):
        for k in range(CK // 16):
            ix = pl.ds(k * 16, 16)
            sv = src_v[j, ix]
            dv = dst_v[j, ix]
            e = plsc.load_gather(as_v, [sv]) + plsc.load_gather(ad_v, [dv])
            e = jnp.where(e >= 0.0, e, e * 0.2)
            ex_v[j, ix] = jnp.exp(e)


def _scatter_ex(ex_v, dst_v, s_sh, xsem):
    @pl.loop(0, NCH2, step=16)
    def _scat(j0):
        for b in range(16):
            pltpu.async_copy(ex_v.at[j0 + b], s_sh.at[dst_v.at[j0 + b]],
                             xsem, add=True)
        for b in range(16):
            pltpu.make_async_copy(ex_v.at[j0 + b],
                                  s_sh.at[dst_v.at[j0 + b]], xsem).wait()


def _zero_rows(rows, D):
    @pl.loop(0, CK)
    def _zrow(e):
        for k in range(D // 16):
            rows[0, e, pl.ds(k * 16, 16)] = jnp.zeros((16,), jnp.float32)


def _pipeline_chunks(D, j_lo, j_hi, h_hbm, src_v, dst_v, al_v, rows, out_sh,
                     gsem, ssem):
    def g_issue(j, b):
        pltpu.async_copy(h_hbm.at[src_v.at[j]], rows.at[b], gsem)

    def g_wait(j, b):
        pltpu.make_async_copy(h_hbm.at[src_v.at[j]], rows.at[b], gsem).wait()

    def s_issue(j, b):
        pltpu.async_copy(rows.at[b], out_sh.at[dst_v.at[j]], ssem, add=True)

    def s_wait(j, b):
        pltpu.make_async_copy(rows.at[b], out_sh.at[dst_v.at[j]], ssem).wait()

    for k in range(3):
        g_issue(j_lo + k, k)

    @pl.loop(j_lo, j_hi, step=4)
    def _steps(j0):
        for b in range(4):
            j = j0 + b
            g_wait(j, b)

            @pl.loop(0, CK // 16)
            def _scale(g):
                av = al_v[j, pl.ds(g * 16, 16)]
                for l in range(16):
                    a = av[l]
                    e = g * 16 + l
                    for k in range(D // 16):
                        ix = pl.ds(k * 16, 16)
                        rows[b, e, ix] = rows[b, e, ix] * a

            @pl.when(j > j_lo)
            def _drain_prev():
                s_wait(j, b)

            s_issue(j, b)

            @pl.when(j + 3 < j_hi)
            def _prefetch():
                g_issue(j + 3, (b + 3) % 4)

    s_wait(j_lo, 0)


def _sc_l1_body(src_hbm, dst_hbm, as_hbm, ad_hbm, h_hbm,
                q00_hbm, q01_hbm, q10_hbm, q11_hbm,
                as_v, ad_v, s_v, src_v, dst_v, ex_v, z_v, rows,
                s_sh, out_sh, gsem, ssem, xsem):
    cid = lax.axis_index("c")
    sid = lax.axis_index("s")

    _zero_slice(z_v, s_sh, sid)

    pltpu.sync_copy(as_hbm, as_v)
    pltpu.sync_copy(ad_hbm, ad_v)
    pltpu.sync_copy(src_hbm.at[sid], src_v)
    pltpu.sync_copy(dst_hbm.at[sid], dst_v)

    _compute_ex(as_v, ad_v, src_v, dst_v, ex_v)
    plsc.subcore_barrier()
    _scatter_ex(ex_v, dst_v, s_sh, xsem)
    plsc.subcore_barrier()
    pltpu.sync_copy(s_sh, s_v)

    off = cid * (2 * NP)

    @pl.loop(0, NCH2)
    def _alpha(j):
        for k in range(CK // 16):
            ix = pl.ds(k * 16, 16)
            dv = dst_v[j, ix]
            sg = plsc.load_gather(s_v, [dv])
            ex_v[j, ix] = ex_v[j, ix] / (sg + 1e-16)
            src_v[j, ix] = src_v[j, ix] + off

    quarters = ((q00_hbm, q10_hbm), (q01_hbm, q11_hbm))
    for p in range(2):
        _zero_rows(rows, DHQ)
        for r in range(RPT // CK):
            pltpu.sync_copy(rows.at[0],
                            out_sh.at[pl.ds(sid * RPT + r * CK, CK)])
        plsc.subcore_barrier()

        _pipeline_chunks(DHQ, 0, NCH2, h_hbm, src_v, dst_v, ex_v, rows,
                         out_sh, gsem, ssem)
        plsc.subcore_barrier()

        qa, qb = quarters[p]
        for r in range(RPT // CK):
            sl = pl.ds(sid * RPT + r * CK, CK)

            @pl.when(cid == 0)
            def _c0():
                pltpu.sync_copy(out_sh.at[sl], qa.at[sl])

            @pl.when(cid == 1)
            def _c1():
                pltpu.sync_copy(out_sh.at[sl], qb.at[sl])

        if p == 0:
            @pl.loop(0, NCH2)
            def _bump(j):
                for k in range(CK // 16):
                    ix = pl.ds(k * 16, 16)
                    src_v[j, ix] = src_v[j, ix] + NP


_sc_l1 = pl.kernel(
    _sc_l1_body,
    out_type=[
        jax.ShapeDtypeStruct((NP, DHQ), jnp.float32),
        jax.ShapeDtypeStruct((NP, DHQ), jnp.float32),
        jax.ShapeDtypeStruct((NP, DHQ), jnp.float32),
        jax.ShapeDtypeStruct((NP, DHQ), jnp.float32),
    ],
    mesh=_mesh,
    compiler_params=_sc_params,
    scratch_types=[
        pltpu.VMEM((NP,), jnp.float32),
        pltpu.VMEM((NP,), jnp.float32),
        pltpu.VMEM((NP,), jnp.float32),
        pltpu.VMEM((NCH2, CK), jnp.int32),
        pltpu.VMEM((NCH2, CK), jnp.int32),
        pltpu.VMEM((NCH2, CK), jnp.float32),
        pltpu.VMEM((RPT,), jnp.float32),
        pltpu.VMEM((4, CK, DHQ), jnp.float32),
        pltpu.VMEM_SHARED((NP,), jnp.float32),
        pltpu.VMEM_SHARED((NP, DHQ), jnp.float32),
        pltpu.SemaphoreType.DMA,
        pltpu.SemaphoreType.DMA,
        pltpu.SemaphoreType.DMA,
    ],
)


def _sc_l2_body(src_hbm, dst_hbm, as_hbm, ad_hbm, h_hbm,
                q0_hbm, q1_hbm,
                as_v, ad_v, s_v, src_v, dst_v, ex_v, z_v, rows,
                s_sh, out_sh, gsem, ssem, xsem):
    cid = lax.axis_index("c")
    sid = lax.axis_index("s")

    _zero_slice(z_v, s_sh, sid)

    pltpu.sync_copy(as_hbm, as_v)
    pltpu.sync_copy(ad_hbm, ad_v)
    pltpu.sync_copy(src_hbm.at[sid], src_v)
    pltpu.sync_copy(dst_hbm.at[sid], dst_v)

    _compute_ex(as_v, ad_v, src_v, dst_v, ex_v)
    plsc.subcore_barrier()
    _scatter_ex(ex_v, dst_v, s_sh, xsem)
    plsc.subcore_barrier()
    pltpu.sync_copy(s_sh, s_v)

    j_lo = cid * NCH
    j_hi = j_lo + NCH

    @pl.loop(j_lo, j_hi)
    def _alpha(j):
        for k in range(CK // 16):
            ix = pl.ds(k * 16, 16)
            dv = dst_v[j, ix]
            sg = plsc.load_gather(s_v, [dv])
            ex_v[j, ix] = ex_v[j, ix] / (sg + 1e-16)

    _zero_rows(rows, D2)
    for r in range(RPT // CK):
        pltpu.sync_copy(rows.at[0], out_sh.at[pl.ds(sid * RPT + r * CK, CK)])
    plsc.subcore_barrier()

    _pipeline_chunks(D2, j_lo, j_hi, h_hbm, src_v, dst_v, ex_v, rows,
                     out_sh, gsem, ssem)
    plsc.subcore_barrier()

    for r in range(RPT // CK):
        sl = pl.ds(sid * RPT + r * CK, CK)

        @pl.when(cid == 0)
        def _c0():
            pltpu.sync_copy(out_sh.at[sl], q0_hbm.at[sl])

        @pl.when(cid == 1)
        def _c1():
            pltpu.sync_copy(out_sh.at[sl], q1_hbm.at[sl])


_sc_l2 = pl.kernel(
    _sc_l2_body,
    out_type=[
        jax.ShapeDtypeStruct((NP, D2), jnp.float32),
        jax.ShapeDtypeStruct((NP, D2), jnp.float32),
    ],
    mesh=_mesh,
    compiler_params=_sc_params,
    scratch_types=[
        pltpu.VMEM((NP,), jnp.float32),
        pltpu.VMEM((NP,), jnp.float32),
        pltpu.VMEM((NP,), jnp.float32),
        pltpu.VMEM((NCH2, CK), jnp.int32),
        pltpu.VMEM((NCH2, CK), jnp.int32),
        pltpu.VMEM((NCH2, CK), jnp.float32),
        pltpu.VMEM((RPT,), jnp.float32),
        pltpu.VMEM((4, CK, D2), jnp.float32),
        pltpu.VMEM_SHARED((NP,), jnp.float32),
        pltpu.VMEM_SHARED((NP, D2), jnp.float32),
        pltpu.SemaphoreType.DMA,
        pltpu.SemaphoreType.DMA,
        pltpu.SemaphoreType.DMA,
    ],
)


def kernel(x, edge_index, W1, a_src1, a_dst1, b1, W2, a_src2, a_dst2, b2):
    src = edge_index[0]
    dst = edge_index[1]
    pad_ids = (jnp.arange(EP - E, dtype=jnp.int32) % (NP - N)) + N
    src16 = jnp.concatenate([src, pad_ids]).reshape(16, NCH2, CK)
    dst16 = jnp.concatenate([dst, pad_ids]).reshape(16, NCH2, CK)

    a1 = jnp.stack([a_src1, a_dst1], axis=1)
    h, aa1 = _tc_lin1(x, W1, a1)
    hp = jnp.pad(h, ((0, NP - N), (0, 0)))
    htab1 = jnp.concatenate([hp[:, k * DHQ:(k + 1) * DHQ] for k in range(4)])
    as1 = jnp.pad(aa1[:, 0], (0, NP - N))
    ad1 = jnp.pad(aa1[:, 1], (0, NP - N))

    q00, q01, q10, q11 = _sc_l1(src16, dst16, as1, ad1, htab1)

    b1r = b1.reshape(1, 128)
    w2p = jnp.pad(W2, ((0, 0), (0, D2 - 2)))
    a2 = jnp.pad(jnp.stack([a_src2, a_dst2], axis=1), ((0, D2 - 2), (0, 0)))
    h2, aa2 = _tc_lin2(q00, q01, q10, q11, b1r, w2p, a2)

    q2a, q2b = _sc_l2(src16, dst16, aa2[:, 0], aa2[:, 1], h2)

    b2r = jnp.pad(b2, (0, D2 - 2)).reshape(1, D2)
    out = _tc_fin(q2a, q2b, b2r)
    return out[:N, :2]


# speedup vs baseline: 1.0527x; 1.0479x over previous
import jax
import jax.numpy as jnp
from jax import lax
from jax.experimental import pallas as pl
from jax.experimental.pallas import tpu as pltpu
from jax.experimental.pallas import tpu_sc as plsc

N = 10000
E = 320000
NP = 10240
NW = 32
CK = 128
NCH = 80
NCH2 = 160
EP = NW * NCH * CK
RPT = NP // 16
DHQ = 32
D2 = 16

_mesh = plsc.VectorSubcoreMesh(core_axis_name="c", subcore_axis_name="s")
_sc_params = pltpu.CompilerParams(
    needs_layout_passes=False, use_tc_tiling_on_sc=False
)


def _tc_lin1_body(x_ref, w_ref, a_ref, h_ref, aa_ref):
    h = jnp.dot(x_ref[...], w_ref[...], preferred_element_type=jnp.float32)
    h_ref[...] = h
    aa_ref[...] = jnp.dot(h, a_ref[...], preferred_element_type=jnp.float32)


def _tc_lin1(x, w, a):
    return pl.pallas_call(
        _tc_lin1_body,
        grid=(10,),
        in_specs=[
            pl.BlockSpec((1000, 128), lambda i: (i, 0)),
            pl.BlockSpec((128, 128), lambda i: (0, 0)),
            pl.BlockSpec((128, 2), lambda i: (0, 0)),
        ],
        out_specs=[
            pl.BlockSpec((1000, 128), lambda i: (i, 0)),
            pl.BlockSpec((1000, 2), lambda i: (i, 0)),
        ],
        out_shape=[
            jax.ShapeDtypeStruct((N, 128), jnp.float32),
            jax.ShapeDtypeStruct((N, 2), jnp.float32),
        ],
    )(x, w, a)


def _tc_lin2_body(p0_ref, p1_ref, p2_ref, p3_ref, b_ref, w_ref, a_ref,
                  h2_ref, aa_ref):
    g = jnp.concatenate(
        [p0_ref[...], p1_ref[...], p2_ref[...], p3_ref[...]], axis=1)
    g = jnp.maximum(g + b_ref[...], 0.0)
    h2 = jnp.dot(g, w_ref[...], preferred_element_type=jnp.float32)
    h2_ref[...] = h2
    aa_ref[...] = jnp.dot(h2, a_ref[...], preferred_element_type=jnp.float32)


def _tc_lin2(p0, p1, p2, p3, b1, w2, a2):
    return pl.pallas_call(
        _tc_lin2_body,
        grid=(10,),
        in_specs=[
            pl.BlockSpec((1024, DHQ), lambda i: (i, 0)),
            pl.BlockSpec((1024, DHQ), lambda i: (i, 0)),
            pl.BlockSpec((1024, DHQ), lambda i: (i, 0)),
            pl.BlockSpec((1024, DHQ), lambda i: (i, 0)),
            pl.BlockSpec((1, 128), lambda i: (0, 0)),
            pl.BlockSpec((128, D2), lambda i: (0, 0)),
            pl.BlockSpec((D2, 2), lambda i: (0, 0)),
        ],
        out_specs=[
            pl.BlockSpec((1024, D2), lambda i: (i, 0)),
            pl.BlockSpec((1024, 2), lambda i: (i, 0)),
        ],
        out_shape=[
            jax.ShapeDtypeStruct((NP, D2), jnp.float32),
            jax.ShapeDtypeStruct((NP, 2), jnp.float32),
        ],
    )(p0, p1, p2, p3, b1, w2, a2)


def _tc_fin_body(q0_ref, q1_ref, b_ref, o_ref):
    o_ref[...] = q0_ref[...] + q1_ref[...] + b_ref[...]


def _tc_fin(q0, q1, b2):
    return pl.pallas_call(
        _tc_fin_body,
        grid=(10,),
        in_specs=[
            pl.BlockSpec((1024, D2), lambda i: (i, 0)),
            pl.BlockSpec((1024, D2), lambda i: (i, 0)),
            pl.BlockSpec((1, D2), lambda i: (0, 0)),
        ],
        out_specs=pl.BlockSpec((1024, D2), lambda i: (i, 0)),
        out_shape=jax.ShapeDtypeStruct((NP, D2), jnp.float32),
    )(q0, q1, b2)


def _zero_slice(z_v, sh, sid):
    @pl.loop(0, RPT // 16)
    def _zero(i):
        z_v[pl.ds(i * 16, 16)] = jnp.zeros((16,), jnp.float32)

    pltpu.sync_copy(z_v, sh.at[pl.ds(sid * RPT, RPT)])


def _zero_rows(rows, D):
    @pl.loop(0, CK)
    def _zrow(e):
        for k in range(D // 16):
            rows[0, e, pl.ds(k * 16, 16)] = jnp.zeros((16,), jnp.float32)


def _combine_s(s_v, t_v):
    @pl.loop(0, NP // 16)
    def _acc(i):
        ix = pl.ds(i * 16, 16)
        s_v[ix] = s_v[ix] + t_v[ix]


def _pipeline_chunks(D, j_lo, j_hi, h_hbm, src_v, dst_v, al_v, rows, out_sh,
                     gsem, ssem):
    def g_issue(j, b):
        pltpu.async_copy(h_hbm.at[src_v.at[j]], rows.at[b], gsem)

    def g_wait(j, b):
        pltpu.make_async_copy(h_hbm.at[src_v.at[j]], rows.at[b], gsem).wait()

    def s_issue(j, b):
        pltpu.async_copy(rows.at[b], out_sh.at[dst_v.at[j]], ssem, add=True)

    def s_wait(j, b):
        pltpu.make_async_copy(rows.at[b], out_sh.at[dst_v.at[j]], ssem).wait()

    for k in range(3):
        g_issue(j_lo + k, k)

    @pl.loop(j_lo, j_hi, step=4)
    def _steps(j0):
        for b in range(4):
            j = j0 + b
            g_wait(j, b)

            @pl.loop(0, CK // 16)
            def _scale(g):
                av = al_v[j, pl.ds(g * 16, 16)]
                for l in range(16):
                    a = av[l]
                    e = g * 16 + l
                    for k in range(D // 16):
                        ix = pl.ds(k * 16, 16)
                        rows[b, e, ix] = rows[b, e, ix] * a

            @pl.when(j >= j_lo + 1)
            def _drain_prev():
                s_wait(j, b)

            s_issue(j, b)

            @pl.when(j + 3 < j_hi)
            def _prefetch():
                g_issue(j + 3, (b + 3) % 4)

    s_wait(j_lo, 0)


def _sc_a_body(src_hbm, dst_hbm, as_hbm, ad_hbm,
               ex_hbm, s0_hbm, s1_hbm,
               as_v, ad_v, src_v, dst_v, ex_v, z_v, s_sh, xsem):
    cid = lax.axis_index("c")
    sid = lax.axis_index("s")
    wid = sid * 2 + cid

    _zero_slice(z_v, s_sh, sid)

    pltpu.sync_copy(as_hbm, as_v)
    pltpu.sync_copy(ad_hbm, ad_v)
    pltpu.sync_copy(src_hbm.at[wid], src_v)
    pltpu.sync_copy(dst_hbm.at[wid], dst_v)

    @pl.loop(0, NCH)
    def _chunk(j):
        for k in range(CK // 16):
            ix = pl.ds(k * 16, 16)
            sv = src_v[j, ix]
            dv = dst_v[j, ix]
            e = plsc.load_gather(as_v, [sv]) + plsc.load_gather(ad_v, [dv])
            e = jnp.where(e >= 0.0, e, e * 0.2)
            ex_v[j, ix] = jnp.exp(e)

    plsc.subcore_barrier()

    @pl.loop(0, NCH, step=16)
    def _scat(j0):
        for b in range(16):
            pltpu.async_copy(ex_v.at[j0 + b], s_sh.at[dst_v.at[j0 + b]],
                             xsem, add=True)
        for b in range(16):
            pltpu.make_async_copy(ex_v.at[j0 + b],
                                  s_sh.at[dst_v.at[j0 + b]], xsem).wait()

    plsc.subcore_barrier()

    pltpu.sync_copy(ex_v, ex_hbm.at[wid])
    sl = pl.ds(sid * RPT, RPT)

    @pl.when(cid == 0)
    def _w0():
        pltpu.sync_copy(s_sh.at[sl], s0_hbm.at[sl])

    @pl.when(cid == 1)
    def _w1():
        pltpu.sync_copy(s_sh.at[sl], s1_hbm.at[sl])


_sc_a = pl.kernel(
    _sc_a_body,
    out_type=[
        jax.ShapeDtypeStruct((NW, NCH, CK), jnp.float32),
        jax.ShapeDtypeStruct((NP,), jnp.float32),
        jax.ShapeDtypeStruct((NP,), jnp.float32),
    ],
    mesh=_mesh,
    compiler_params=_sc_params,
    scratch_types=[
        pltpu.VMEM((NP,), jnp.float32),
        pltpu.VMEM((NP,), jnp.float32),
        pltpu.VMEM((NCH, CK), jnp.int32),
        pltpu.VMEM((NCH, CK), jnp.int32),
        pltpu.VMEM((NCH, CK), jnp.float32),
        pltpu.VMEM((RPT,), jnp.float32),
        pltpu.VMEM_SHARED((NP,), jnp.float32),
        pltpu.SemaphoreType.DMA,
    ],
)


def _sc_b1_body(src_hbm, dst_hbm, ex_hbm, s0_hbm, s1_hbm, h_hbm,
                q00_hbm, q01_hbm, q10_hbm, q11_hbm,
                s_v, t_v, src_v, dst_v, al_v, rows, out_sh, gsem, ssem):
    cid = lax.axis_index("c")
    sid = lax.axis_index("s")

    pltpu.sync_copy(s0_hbm, s_v)
    pltpu.sync_copy(s1_hbm, t_v)
    _combine_s(s_v, t_v)

    pltpu.sync_copy(src_hbm.at[sid], src_v)
    pltpu.sync_copy(dst_hbm.at[sid], dst_v)
    pltpu.sync_copy(ex_hbm.at[sid], al_v)

    off = cid * (2 * NP)

    @pl.loop(0, NCH2)
    def _alpha(j):
        for k in range(CK // 16):
            ix = pl.ds(k * 16, 16)
            dv = dst_v[j, ix]
            sg = plsc.load_gather(s_v, [dv])
            al_v[j, ix] = al_v[j, ix] / (sg + 1e-16)
            src_v[j, ix] = src_v[j, ix] + off

    quarters = ((q00_hbm, q10_hbm), (q01_hbm, q11_hbm))
    for p in range(2):
        _zero_rows(rows, DHQ)
        for r in range(RPT // CK):
            pltpu.sync_copy(rows.at[0],
                            out_sh.at[pl.ds(sid * RPT + r * CK, CK)])
        plsc.subcore_barrier()

        _pipeline_chunks(DHQ, 0, NCH2, h_hbm, src_v, dst_v, al_v, rows,
                         out_sh, gsem, ssem)
        plsc.subcore_barrier()

        qa, qb = quarters[p]
        for r in range(RPT // CK):
            sl = pl.ds(sid * RPT + r * CK, CK)

            @pl.when(cid == 0)
            def _c0():
                pltpu.sync_copy(out_sh.at[sl], qa.at[sl])

            @pl.when(cid == 1)
            def _c1():
                pltpu.sync_copy(out_sh.at[sl], qb.at[sl])

        if p == 0:
            @pl.loop(0, NCH2)
            def _bump(j):
                for k in range(CK // 16):
                    ix = pl.ds(k * 16, 16)
                    src_v[j, ix] = src_v[j, ix] + NP


_sc_b1 = pl.kernel(
    _sc_b1_body,
    out_type=[
        jax.ShapeDtypeStruct((NP, DHQ), jnp.float32),
        jax.ShapeDtypeStruct((NP, DHQ), jnp.float32),
        jax.ShapeDtypeStruct((NP, DHQ), jnp.float32),
        jax.ShapeDtypeStruct((NP, DHQ), jnp.float32),
    ],
    mesh=_mesh,
    compiler_params=_sc_params,
    scratch_types=[
        pltpu.VMEM((NP,), jnp.float32),
        pltpu.VMEM((NP,), jnp.float32),
        pltpu.VMEM((NCH2, CK), jnp.int32),
        pltpu.VMEM((NCH2, CK), jnp.int32),
        pltpu.VMEM((NCH2, CK), jnp.float32),
        pltpu.VMEM((4, CK, DHQ), jnp.float32),
        pltpu.VMEM_SHARED((NP, DHQ), jnp.float32),
        pltpu.SemaphoreType.DMA,
        pltpu.SemaphoreType.DMA,
    ],
)


def _sc_b2_body(src_hbm, dst_hbm, ex_hbm, s0_hbm, s1_hbm, h_hbm,
                q0_hbm, q1_hbm,
                s_v, t_v, src_v, dst_v, al_v, rows, out_sh, gsem, ssem):
    cid = lax.axis_index("c")
    sid = lax.axis_index("s")
    wid = sid * 2 + cid

    pltpu.sync_copy(s0_hbm, s_v)
    pltpu.sync_copy(s1_hbm, t_v)
    _combine_s(s_v, t_v)

    pltpu.sync_copy(src_hbm.at[wid], src_v)
    pltpu.sync_copy(dst_hbm.at[wid], dst_v)
    pltpu.sync_copy(ex_hbm.at[wid], al_v)

    @pl.loop(0, NCH)
    def _alpha(j):
        for k in range(CK // 16):
            ix = pl.ds(k * 16, 16)
            dv = dst_v[j, ix]
            sg = plsc.load_gather(s_v, [dv])
            al_v[j, ix] = al_v[j, ix] / (sg + 1e-16)

    _zero_rows(rows, D2)
    for r in range(RPT // CK):
        pltpu.sync_copy(rows.at[0], out_sh.at[pl.ds(sid * RPT + r * CK, CK)])
    plsc.subcore_barrier()

    _pipeline_chunks(D2, 0, NCH, h_hbm, src_v, dst_v, al_v, rows,
                     out_sh, gsem, ssem)
    plsc.subcore_barrier()

    for r in range(RPT // CK):
        sl = pl.ds(sid * RPT + r * CK, CK)

        @pl.when(cid == 0)
        def _c0():
            pltpu.sync_copy(out_sh.at[sl], q0_hbm.at[sl])

        @pl.when(cid == 1)
        def _c1():
            pltpu.sync_copy(out_sh.at[sl], q1_hbm.at[sl])


_sc_b2 = pl.kernel(
    _sc_b2_body,
    out_type=[
        jax.ShapeDtypeStruct((NP, D2), jnp.float32),
        jax.ShapeDtypeStruct((NP, D2), jnp.float32),
    ],
    mesh=_mesh,
    compiler_params=_sc_params,
    scratch_types=[
        pltpu.VMEM((NP,), jnp.float32),
        pltpu.VMEM((NP,), jnp.float32),
        pltpu.VMEM((NCH, CK), jnp.int32),
        pltpu.VMEM((NCH, CK), jnp.int32),
        pltpu.VMEM((NCH, CK), jnp.float32),
        pltpu.VMEM((4, CK, D2), jnp.float32),
        pltpu.VMEM_SHARED((NP, D2), jnp.float32),
        pltpu.SemaphoreType.DMA,
        pltpu.SemaphoreType.DMA,
    ],
)


def kernel(x, edge_index, W1, a_src1, a_dst1, b1, W2, a_src2, a_dst2, b2):
    src = edge_index[0]
    dst = edge_index[1]
    pad_ids = (jnp.arange(EP - E, dtype=jnp.int32) % (NP - N)) + N
    src_p = jnp.concatenate([src, pad_ids])
    dst_p = jnp.concatenate([dst, pad_ids])
    src32 = src_p.reshape(NW, NCH, CK)
    dst32 = dst_p.reshape(NW, NCH, CK)
    src16 = src_p.reshape(16, NCH2, CK)
    dst16 = dst_p.reshape(16, NCH2, CK)

    a1 = jnp.stack([a_src1, a_dst1], axis=1)
    h, aa1 = _tc_lin1(x, W1, a1)
    hp = jnp.pad(h, ((0, NP - N), (0, 0)))
    htab1 = jnp.concatenate([hp[:, k * DHQ:(k + 1) * DHQ] for k in range(4)])
    as1 = jnp.pad(aa1[:, 0], (0, NP - N))
    ad1 = jnp.pad(aa1[:, 1], (0, NP - N))

    ex1, s1a, s1b = _sc_a(src32, dst32, as1, ad1)
    ex16 = ex1.reshape(16, NCH2, CK)
    q00, q01, q10, q11 = _sc_b1(src16, dst16, ex16, s1a, s1b, htab1)

    b1r = b1.reshape(1, 128)
    w2p = jnp.pad(W2, ((0, 0), (0, D2 - 2)))
    a2 = jnp.pad(jnp.stack([a_src2, a_dst2], axis=1), ((0, D2 - 2), (0, 0)))
    h2, aa2 = _tc_lin2(q00, q01, q10, q11, b1r, w2p, a2)

    ex2, s2a, s2b = _sc_a(src32, dst32, aa2[:, 0], aa2[:, 1])
    q2a, q2b = _sc_b2(src32, dst32, ex2, s2a, s2b, h2)

    b2r = jnp.pad(b2, (0, D2 - 2)).reshape(1, D2)
    out = _tc_fin(q2a, q2b, b2r)
    return out[:N, :2]
